# baseline (speedup 1.0000x reference)
import functools

import jax
import jax.numpy as jnp
from jax import lax
from jax.experimental import pallas as pl
from jax.experimental.pallas import tpu as pltpu
from jax.experimental.pallas import tpu_sc as plsc

_L = 16


def _sim_a_kernel(scal_ref, x_ref, y_ref, mem_ref, sim_ref, yidx_ref, *,
                  n_pos):
    xb = x_ref[...]
    mb = mem_ref[...]
    sim = jax.lax.dot_general(xb, mb, (((1,), (1,)), ((), ())))
    sim_ref[...] = sim
    group_scale = scal_ref[0]
    cols = jax.lax.broadcasted_iota(jnp.int32, (n_pos, n_pos), 1)
    boosted = sim + jnp.where(cols == y_ref[...], 2.0 * group_scale, 0.0)
    yidx_ref[...] = jnp.argmax(boosted, axis=1).astype(jnp.int32)[:, None]


def _sim_b_kernel(x_ref, mem_ref, alias_ref, sim_ref):
    del alias_ref
    xb = x_ref[...]
    mb = mem_ref[...]
    sim_ref[...] = jax.lax.dot_general(xb, mb, (((1,), (1,)), ((), ())))


def _upd_kernel(scal_ref, get_ref, cnt_ref, vis_ref, mem_ref, alias_ref,
                out_ref):
    del alias_ref
    momentum = scal_ref[3]
    eps = scal_ref[4]
    counts = cnt_ref[:, 0:1]
    vis = vis_ref[:, 0:1]
    get = get_ref[...]
    mb = mem_ref[...]
    valid = jnp.where((counts > 0.1) & (vis > 0.5), 1.0, 0.0)
    keep = valid * momentum + 1.0 - valid
    blend = (1.0 - momentum) * valid / (counts + eps)
    upd = mb * keep + get * blend
    nrm = jnp.maximum(jnp.sqrt(jnp.sum(upd * upd, axis=1, keepdims=True)),
                      1e-12)
    out_ref[...] = upd / nrm


def _make_sc_kernel(n_pos, d, M, n_x):
    ns = 16
    rpw = n_pos // ns
    n_neg_rows = n_x - n_pos
    xpw = n_neg_rows // ns
    bank_rows = M - 2 * n_pos
    bpw = bank_rows // ns
    mesh = plsc.VectorSubcoreMesh(core_axis_name="c", subcore_axis_name="s")

    @functools.partial(
        pl.kernel, mesh=mesh,
        out_type=[
            jax.ShapeDtypeStruct((M, d), jnp.float32),
            jax.ShapeDtypeStruct((n_pos, d), jnp.float32),
            jax.ShapeDtypeStruct((n_pos, _L), jnp.float32),
            jax.ShapeDtypeStruct((n_pos, _L), jnp.float32),
        ],
        scratch_types=[
            pltpu.VMEM((n_pos // 16,), jnp.int32),
            pltpu.VMEM((n_pos // 16, 64), jnp.float32),
            pltpu.VMEM((n_pos // 16,), jnp.int32),
            pltpu.VMEM((n_pos // 16, _L), jnp.float32),
            pltpu.VMEM(((M - 2 * n_pos) // 16, 64), jnp.float32),
            pltpu.VMEM_SHARED((n_pos, 64), jnp.float32),
            pltpu.VMEM_SHARED((n_pos, _L), jnp.float32),
            pltpu.VMEM_SHARED((n_pos, _L), jnp.float32),
        ],
    )
    def sc_kernel(yidx_hbm, x_hbm, vis_hbm, mem_hbm, zeros_hbm,
                  zerosn_hbm, ones_hbm, newmem_hbm, get_hbm, cnt_hbm, visq_hbm,
                  yv, xv, vv, ones, bankbuf, sget, scnt, svis):
        cid = lax.axis_index("c")
        sid = lax.axis_index("s")

        @pl.when(cid == 0)
        def _scatter_phase():
            base = sid * rpw
            pltpu.sync_copy(yidx_hbm.at[pl.ds(base, rpw)], yv)
            pltpu.sync_copy(x_hbm.at[pl.ds(base, rpw)], xv)
            pltpu.sync_copy(vis_hbm.at[pl.ds(base, rpw)], vv)
            pltpu.sync_copy(zeros_hbm.at[pl.ds(base, rpw)],
                            sget.at[pl.ds(base, rpw)])
            pltpu.sync_copy(zerosn_hbm.at[pl.ds(base, rpw)],
                            scnt.at[pl.ds(base, rpw)])
            pltpu.sync_copy(zerosn_hbm.at[pl.ds(base, rpw)],
                            svis.at[pl.ds(base, rpw)])
            pltpu.sync_copy(ones_hbm.at[pl.ds(base, rpw)], ones)
            plsc.subcore_barrier()
            pltpu.sync_copy(xv, sget.at[yv], add=True)
            pltpu.sync_copy(ones, scnt.at[yv], add=True)
            pltpu.sync_copy(ones, svis.at[vv], add=True)
            plsc.subcore_barrier()
            pltpu.sync_copy(sget.at[pl.ds(base, rpw)],
                            get_hbm.at[pl.ds(base, rpw)])
            pltpu.sync_copy(scnt.at[pl.ds(base, rpw)],
                            cnt_hbm.at[pl.ds(base, rpw)])
            pltpu.sync_copy(svis.at[pl.ds(base, rpw)],
                            visq_hbm.at[pl.ds(base, rpw)])

        @pl.when(cid == 1)
        def _route_phase():
            start = n_pos
            pltpu.sync_copy(x_hbm.at[pl.ds(n_pos + sid * xpw, xpw)], xv)
            pltpu.sync_copy(xv, newmem_hbm.at[pl.ds(pl.multiple_of(start + sid * xpw, 8), xpw)])
            tail = start + n_neg_rows
            tbase = pl.multiple_of(tail + sid * bpw, 8)
            pltpu.sync_copy(mem_hbm.at[pl.ds(tbase, bpw)], bankbuf)
            pltpu.sync_copy(bankbuf, newmem_hbm.at[pl.ds(tbase, bpw)])

    return sc_kernel


def kernel(x, y, visible, n_pos, n_neg, lru, memory, params, eps):
    n_pos_static = visible.shape[1]
    M, d = memory.shape
    tile = n_pos_static
    n_tiles = M // tile
    scal = jnp.concatenate(
        [params.astype(jnp.float32), jnp.asarray(eps, jnp.float32)[None]])
    x_pos = x[:n_pos_static]
    y2 = y.astype(jnp.int32)[:, None]
    vis_flat = visible.astype(jnp.int32).reshape(-1)

    sim_a, yidx = pl.pallas_call(
        functools.partial(_sim_a_kernel, n_pos=n_pos_static),
        grid=(1,),
        in_specs=[
            pl.BlockSpec(memory_space=pltpu.SMEM),
            pl.BlockSpec((n_pos_static, d), lambda j: (0, 0)),
            pl.BlockSpec((n_pos_static, 1), lambda j: (0, 0)),
            pl.BlockSpec((tile, d), lambda j: (0, 0)),
        ],
        out_specs=[
            pl.BlockSpec((n_pos_static, tile), lambda j: (0, 0)),
            pl.BlockSpec((n_pos_static, 1), lambda j: (0, 0)),
        ],
        out_shape=[
            jax.ShapeDtypeStruct((n_pos_static, M), jnp.float32),
            jax.ShapeDtypeStruct((n_pos_static, 1), jnp.int32),
        ],
    )(scal, x_pos, y2, memory)
    y_idx = yidx.reshape(n_pos_static)

    zeros_t = jnp.zeros((n_pos_static, d), jnp.float32)
    zerosn_t = jnp.zeros((n_pos_static, _L), jnp.float32)
    ones_t = jnp.tile(
        (jnp.arange(_L) == 0).astype(jnp.float32)[None, :], (n_pos_static, 1))
    sc = _make_sc_kernel(n_pos_static, d, M, x.shape[0])
    newmem0, get, cnt, visq = sc(y_idx, x, vis_flat, memory,
                                 zeros_t, zerosn_t, ones_t)

    sim = pl.pallas_call(
        _sim_b_kernel,
        grid=(n_tiles - 1,),
        in_specs=[
            pl.BlockSpec((n_pos_static, d), lambda j: (0, 0)),
            pl.BlockSpec((tile, d), lambda j: (j + 1, 0)),
            pl.BlockSpec(memory_space=pl.ANY),
        ],
        out_specs=pl.BlockSpec((n_pos_static, tile), lambda j: (0, j + 1)),
        out_shape=jax.ShapeDtypeStruct((n_pos_static, M), jnp.float32),
        input_output_aliases={2: 0},
    )(x_pos, memory, sim_a)

    new_memory = pl.pallas_call(
        _upd_kernel,
        grid=(1,),
        in_specs=[
            pl.BlockSpec(memory_space=pltpu.SMEM),
            pl.BlockSpec((n_pos_static, d), lambda j: (0, 0)),
            pl.BlockSpec((n_pos_static, _L), lambda j: (0, 0)),
            pl.BlockSpec((n_pos_static, _L), lambda j: (0, 0)),
            pl.BlockSpec((n_pos_static, d), lambda j: (0, 0)),
            pl.BlockSpec(memory_space=pl.ANY),
        ],
        out_specs=pl.BlockSpec((n_pos_static, d), lambda j: (0, 0)),
        out_shape=jax.ShapeDtypeStruct((M, d), jnp.float32),
        input_output_aliases={5: 0},
    )(scal, get, cnt, visq, memory, newmem0)

    return (sim, y_idx, new_memory)


# speedup vs baseline: 1.0275x; 1.0275x over previous
import functools

import jax
import jax.numpy as jnp
from jax import lax
from jax.experimental import pallas as pl
from jax.experimental.pallas import tpu as pltpu
from jax.experimental.pallas import tpu_sc as plsc

_L = 16


def _sim_a_kernel(scal_ref, x_ref, y_ref, mem_ref, sim_ref, yidx_ref, *,
                  n_pos):
    xb = x_ref[...]
    mb = mem_ref[...]
    sim = jax.lax.dot_general(xb, mb, (((1,), (1,)), ((), ())))
    sim_ref[...] = sim
    group_scale = scal_ref[0]
    cols = jax.lax.broadcasted_iota(jnp.int32, (n_pos, n_pos), 1)
    boosted = sim + jnp.where(cols == y_ref[...], 2.0 * group_scale, 0.0)
    yidx_ref[...] = jnp.argmax(boosted, axis=1).astype(jnp.int32)


def _sim_b_kernel(x_ref, mem_ref, alias_ref, sim_ref):
    del alias_ref
    xb = x_ref[...]
    mb = mem_ref[...]
    sim_ref[...] = jax.lax.dot_general(xb, mb, (((1,), (1,)), ((), ())))


def _upd_kernel(scal_ref, get_ref, cnt_ref, vis_ref, mem_ref, alias_ref,
                out_ref):
    del alias_ref
    momentum = scal_ref[3]
    eps = scal_ref[4]
    counts = cnt_ref[:, 0:1]
    vis = vis_ref[:, 0:1]
    get = get_ref[...]
    mb = mem_ref[...]
    valid = jnp.where((counts > 0.1) & (vis > 0.5), 1.0, 0.0)
    keep = valid * momentum + 1.0 - valid
    blend = (1.0 - momentum) * valid / (counts + eps)
    upd = mb * keep + get * blend
    nrm = jnp.maximum(jnp.sqrt(jnp.sum(upd * upd, axis=1, keepdims=True)),
                      1e-12)
    out_ref[...] = upd / nrm


def _make_sc_kernel(n_pos, d, M, n_x):
    ns = 16
    rpw = n_pos // ns
    n_neg_rows = n_x - n_pos
    xpw = n_neg_rows // ns
    bank_rows = M - 2 * n_pos
    bpw = bank_rows // ns
    mesh = plsc.VectorSubcoreMesh(core_axis_name="c", subcore_axis_name="s")

    @functools.partial(
        pl.kernel, mesh=mesh,
        compiler_params=pltpu.CompilerParams(use_tc_tiling_on_sc=True),
        out_type=[
            jax.ShapeDtypeStruct((M, d), jnp.float32),
            jax.ShapeDtypeStruct((n_pos, d), jnp.float32),
            jax.ShapeDtypeStruct((n_pos, _L), jnp.float32),
            jax.ShapeDtypeStruct((n_pos, _L), jnp.float32),
        ],
        scratch_types=[
            pltpu.VMEM((n_pos // 16,), jnp.int32),
            pltpu.VMEM((n_pos // 16, 64), jnp.float32),
            pltpu.VMEM((n_pos // 16,), jnp.int32),
            pltpu.VMEM((n_pos // 16, _L), jnp.float32),
            pltpu.VMEM(((M - 2 * n_pos) // 16, 64), jnp.float32),
            pltpu.VMEM_SHARED((n_pos, 64), jnp.float32),
            pltpu.VMEM_SHARED((n_pos, _L), jnp.float32),
            pltpu.VMEM_SHARED((n_pos, _L), jnp.float32),
        ],
    )
    def sc_kernel(yidx_hbm, x_hbm, vis_hbm, mem_hbm, zeros_hbm,
                  zerosn_hbm, ones_hbm, newmem_hbm, get_hbm, cnt_hbm, visq_hbm,
                  yv, xv, vv, ones, bankbuf, sget, scnt, svis):
        cid = lax.axis_index("c")
        sid = lax.axis_index("s")

        @pl.when(cid == 0)
        def _scatter_phase():
            base = sid * rpw
            pltpu.sync_copy(yidx_hbm.at[pl.ds(base, rpw)], yv)
            pltpu.sync_copy(x_hbm.at[pl.ds(base, rpw)], xv)
            pltpu.sync_copy(vis_hbm.at[pl.ds(base, rpw)], vv)
            pltpu.sync_copy(zeros_hbm.at[pl.ds(base, rpw)],
                            sget.at[pl.ds(base, rpw)])
            pltpu.sync_copy(zerosn_hbm.at[pl.ds(base, rpw)],
                            scnt.at[pl.ds(base, rpw)])
            pltpu.sync_copy(zerosn_hbm.at[pl.ds(base, rpw)],
                            svis.at[pl.ds(base, rpw)])
            pltpu.sync_copy(ones_hbm.at[pl.ds(base, rpw)], ones)
            plsc.subcore_barrier()
            pltpu.sync_copy(xv, sget.at[yv], add=True)
            pltpu.sync_copy(ones, scnt.at[yv], add=True)
            pltpu.sync_copy(ones, svis.at[vv], add=True)
            plsc.subcore_barrier()
            pltpu.sync_copy(sget.at[pl.ds(base, rpw)],
                            get_hbm.at[pl.ds(base, rpw)])
            pltpu.sync_copy(scnt.at[pl.ds(base, rpw)],
                            cnt_hbm.at[pl.ds(base, rpw)])
            pltpu.sync_copy(svis.at[pl.ds(base, rpw)],
                            visq_hbm.at[pl.ds(base, rpw)])

        @pl.when(cid == 1)
        def _route_phase():
            start = n_pos
            pltpu.sync_copy(x_hbm.at[pl.ds(n_pos + sid * xpw, xpw)], xv)
            pltpu.sync_copy(xv, newmem_hbm.at[pl.ds(pl.multiple_of(start + sid * xpw, 8), xpw)])
            tail = start + n_neg_rows
            tbase = pl.multiple_of(tail + sid * bpw, 8)
            pltpu.sync_copy(mem_hbm.at[pl.ds(tbase, bpw)], bankbuf)
            pltpu.sync_copy(bankbuf, newmem_hbm.at[pl.ds(tbase, bpw)])

    return sc_kernel


def kernel(x, y, visible, n_pos, n_neg, lru, memory, params, eps):
    n_pos_static = visible.shape[1]
    M, d = memory.shape
    tile = n_pos_static
    n_tiles = M // tile
    scal = jnp.concatenate(
        [params.astype(jnp.float32), jnp.asarray(eps, jnp.float32)[None]])
    x_pos = x[:n_pos_static]
    y2 = y.astype(jnp.int32)[:, None]
    vis_flat = visible.astype(jnp.int32).reshape(-1)

    sim_a, yidx = pl.pallas_call(
        functools.partial(_sim_a_kernel, n_pos=n_pos_static),
        grid=(1,),
        in_specs=[
            pl.BlockSpec(memory_space=pltpu.SMEM),
            pl.BlockSpec((n_pos_static, d), lambda j: (0, 0)),
            pl.BlockSpec((n_pos_static, 1), lambda j: (0, 0)),
            pl.BlockSpec((tile, d), lambda j: (0, 0)),
        ],
        out_specs=[
            pl.BlockSpec((n_pos_static, tile), lambda j: (0, 0)),
            pl.BlockSpec((n_pos_static,), lambda j: (0,)),
        ],
        out_shape=[
            jax.ShapeDtypeStruct((n_pos_static, M), jnp.float32),
            jax.ShapeDtypeStruct((n_pos_static,), jnp.int32),
        ],
    )(scal, x_pos, y2, memory)
    y_idx = yidx

    zeros_t = jnp.zeros((n_pos_static, d), jnp.float32)
    zerosn_t = jnp.zeros((n_pos_static, _L), jnp.float32)
    ones_t = jnp.tile(
        (jnp.arange(_L) == 0).astype(jnp.float32)[None, :], (n_pos_static, 1))
    sc = _make_sc_kernel(n_pos_static, d, M, x.shape[0])
    newmem0, get, cnt, visq = sc(y_idx, x, vis_flat, memory,
                                 zeros_t, zerosn_t, ones_t)

    sim = pl.pallas_call(
        _sim_b_kernel,
        grid=(n_tiles - 1,),
        in_specs=[
            pl.BlockSpec((n_pos_static, d), lambda j: (0, 0)),
            pl.BlockSpec((tile, d), lambda j: (j + 1, 0)),
            pl.BlockSpec(memory_space=pl.ANY),
        ],
        out_specs=pl.BlockSpec((n_pos_static, tile), lambda j: (0, j + 1)),
        out_shape=jax.ShapeDtypeStruct((n_pos_static, M), jnp.float32),
        input_output_aliases={2: 0},
    )(x_pos, memory, sim_a)

    new_memory = pl.pallas_call(
        _upd_kernel,
        grid=(1,),
        in_specs=[
            pl.BlockSpec(memory_space=pltpu.SMEM),
            pl.BlockSpec((n_pos_static, d), lambda j: (0, 0)),
            pl.BlockSpec((n_pos_static, _L), lambda j: (0, 0)),
            pl.BlockSpec((n_pos_static, _L), lambda j: (0, 0)),
            pl.BlockSpec((n_pos_static, d), lambda j: (0, 0)),
            pl.BlockSpec(memory_space=pl.ANY),
        ],
        out_specs=pl.BlockSpec((n_pos_static, d), lambda j: (0, 0)),
        out_shape=jax.ShapeDtypeStruct((M, d), jnp.float32),
        input_output_aliases={5: 0},
    )(scal, get, cnt, visq, memory, newmem0)

    return (sim, y_idx, new_memory)


# speedup vs baseline: 1.8158x; 1.7673x over previous
import functools

import jax
import jax.numpy as jnp
from jax.experimental import pallas as pl
from jax.experimental.pallas import tpu as pltpu


def _sim_a_kernel(scal_ref, x_ref, y_ref, vis_ref, mem_ref,
                  sim_ref, yidx_ref, upd_ref, *, n_pos):
    xb = x_ref[...]
    mb = mem_ref[...]
    sim = jax.lax.dot_general(xb, mb, (((1,), (1,)), ((), ())))
    sim_ref[...] = sim
    group_scale = scal_ref[0]
    momentum = scal_ref[3]
    eps = scal_ref[4]
    cols = jax.lax.broadcasted_iota(jnp.int32, (n_pos, n_pos), 1)
    boosted = sim + jnp.where(cols == y_ref[...][:, None],
                              2.0 * group_scale, 0.0)
    y_idx = jnp.argmax(boosted, axis=1).astype(jnp.int32)
    yidx_ref[...] = y_idx
    oh = (cols == y_idx[:, None]).astype(jnp.float32)
    get = jax.lax.dot_general(oh, xb, (((0,), (0,)), ((), ())))
    counts = jnp.sum(oh, axis=0)[:, None]
    vis = jnp.max((cols == vis_ref[...]).astype(jnp.float32),
                  axis=0)[:, None]
    valid = jnp.where((counts > 0.1) & (vis > 0.5), 1.0, 0.0)
    keep = valid * momentum + 1.0 - valid
    blend = (1.0 - momentum) * valid / (counts + eps)
    upd = mb * keep + get * blend
    nrm = jnp.maximum(
        jnp.sqrt(jnp.sum(upd * upd, axis=1, keepdims=True)), 1e-12)
    upd_ref[...] = upd / nrm


def _sim_b_kernel(x_ref, mem_ref, alias_ref, sim_ref):
    del alias_ref
    xb = x_ref[...]
    mb = mem_ref[...]
    sim_ref[...] = jax.lax.dot_general(xb, mb, (((1,), (1,)), ((), ())))


def kernel(x, y, visible, n_pos, n_neg, lru, memory, params, eps):
    n_pos_static = visible.shape[1]
    M, d = memory.shape
    tile = n_pos_static
    n_tiles = M // tile
    scal = jnp.concatenate(
        [params.astype(jnp.float32), jnp.asarray(eps, jnp.float32)[None]])
    x_pos = x[:n_pos_static]
    y = y.astype(jnp.int32)
    visible_i = visible.astype(jnp.int32)

    sim_a, y_idx, upd = pl.pallas_call(
        functools.partial(_sim_a_kernel, n_pos=n_pos_static),
        grid=(1,),
        in_specs=[
            pl.BlockSpec(memory_space=pltpu.SMEM),
            pl.BlockSpec((n_pos_static, d), lambda j: (0, 0)),
            pl.BlockSpec((n_pos_static,), lambda j: (0,)),
            pl.BlockSpec((1, n_pos_static), lambda j: (0, 0)),
            pl.BlockSpec((tile, d), lambda j: (0, 0)),
        ],
        out_specs=[
            pl.BlockSpec((n_pos_static, tile), lambda j: (0, 0)),
            pl.BlockSpec((n_pos_static,), lambda j: (0,)),
            pl.BlockSpec((n_pos_static, d), lambda j: (0, 0)),
        ],
        out_shape=[
            jax.ShapeDtypeStruct((n_pos_static, M), jnp.float32),
            jax.ShapeDtypeStruct((n_pos_static,), jnp.int32),
            jax.ShapeDtypeStruct((n_pos_static, d), jnp.float32),
        ],
    )(scal, x_pos, y, visible_i, memory)

    sim = pl.pallas_call(
        _sim_b_kernel,
        grid=(n_tiles - 1,),
        in_specs=[
            pl.BlockSpec((n_pos_static, d), lambda j: (0, 0)),
            pl.BlockSpec((tile, d), lambda j: (j + 1, 0)),
            pl.BlockSpec(memory_space=pl.ANY),
        ],
        out_specs=pl.BlockSpec((n_pos_static, tile), lambda j: (0, j + 1)),
        out_shape=jax.ShapeDtypeStruct((n_pos_static, M), jnp.float32),
        input_output_aliases={2: 0},
    )(x_pos, memory, sim_a)

    new_memory = memory.at[:n_pos_static].set(upd)
    start = n_pos + lru * n_neg
    new_memory = jax.lax.dynamic_update_slice(
        new_memory, x[n_pos_static:], (start, 0))

    return (sim, y_idx, new_memory)
